# speedup vs baseline: 2.1432x; 2.1432x over previous
import functools

import jax
import jax.numpy as jnp
from jax import lax
from jax.experimental import pallas as pl
from jax.experimental.pallas import tpu as pltpu
from jax.experimental.pallas import tpu_sc as plsc

LENGTH = 4096
BATCH = 16
D = 512

NC = 2
NS = 16
NW = NC * NS
ROWS = BATCH * LENGTH
RPW = ROWS // NW
CHUNK = 64
NCHUNK = RPW // CHUNK


def _sc_shuffle(x_flat, ids):
    mesh = plsc.VectorSubcoreMesh(core_axis_name="c", subcore_axis_name="s")

    @functools.partial(
        pl.kernel,
        mesh=mesh,
        out_type=jax.ShapeDtypeStruct((ROWS, D), jnp.float32),
        scratch_types=[
            pltpu.VMEM((CHUNK,), jnp.int32),
            pltpu.VMEM((CHUNK, D), jnp.float32),
            pltpu.SemaphoreType.DMA,
        ],
    )
    def k(x_hbm, ids_hbm, out_hbm, idx_v, rows_v, sem):
        wid = lax.axis_index("s") * NC + lax.axis_index("c")
        base = wid * RPW
        i0 = lax.rem(base, LENGTH)
        b_off = base - i0

        def body(c, carry):
            r0 = base + c * CHUNK
            pltpu.sync_copy(ids_hbm.at[pl.ds(i0 + c * CHUNK, CHUNK)], idx_v)
            for j in range(CHUNK // 16):
                sl = pl.ds(j * 16, 16)
                idx_v[sl] = idx_v[sl] + b_off
            pltpu.async_copy(x_hbm.at[idx_v], rows_v, sem).wait()
            pltpu.sync_copy(rows_v, out_hbm.at[pl.ds(r0, CHUNK)])
            return carry

        lax.fori_loop(0, NCHUNK, body, 0)

    return k(x_flat, ids)


def kernel(inputs, ids_shuffle):
    x_flat = inputs.reshape(ROWS, D)
    ids = ids_shuffle.astype(jnp.int32)
    out = _sc_shuffle(x_flat, ids)
    return out.reshape(BATCH, LENGTH, D)


# speedup vs baseline: 2.8512x; 1.3304x over previous
import functools

import jax
import jax.numpy as jnp
from jax import lax
from jax.experimental import pallas as pl
from jax.experimental.pallas import tpu as pltpu
from jax.experimental.pallas import tpu_sc as plsc

LENGTH = 4096
BATCH = 16
D = 512

NC = 2
NS = 16
NW = NC * NS
ROWS = BATCH * LENGTH
RPW = ROWS // NW
CH = 32
NCHUNK = RPW // CH
NBUF = 4
NG = NCHUNK // NBUF


def _sc_shuffle(x_flat, ids):
    mesh = plsc.VectorSubcoreMesh(core_axis_name="c", subcore_axis_name="s")

    @functools.partial(
        pl.kernel,
        mesh=mesh,
        out_type=jax.ShapeDtypeStruct((ROWS, D), jnp.float32),
        scratch_types=[
            pltpu.VMEM((RPW,), jnp.int32),
            pltpu.VMEM((CH, D), jnp.float32),
            pltpu.VMEM((CH, D), jnp.float32),
            pltpu.VMEM((CH, D), jnp.float32),
            pltpu.VMEM((CH, D), jnp.float32),
            pltpu.SemaphoreType.DMA,
            pltpu.SemaphoreType.DMA,
            pltpu.SemaphoreType.DMA,
            pltpu.SemaphoreType.DMA,
            pltpu.SemaphoreType.DMA,
            pltpu.SemaphoreType.DMA,
            pltpu.SemaphoreType.DMA,
            pltpu.SemaphoreType.DMA,
        ],
    )
    def k(x_hbm, ids_hbm, out_hbm, idx_all, r0, r1, r2, r3,
          g0, g1, g2, g3, w0, w1, w2, w3):
        rows = (r0, r1, r2, r3)
        gsem = (g0, g1, g2, g3)
        wsem = (w0, w1, w2, w3)
        wid = lax.axis_index("s") * NC + lax.axis_index("c")
        base = wid * RPW
        i0 = lax.rem(base, LENGTH)
        b_off = base - i0

        pltpu.sync_copy(ids_hbm.at[pl.ds(i0, RPW)], idx_all)

        def addoff(j, carry):
            sl = pl.ds(j * 16, 16)
            idx_all[sl] = idx_all[sl] + b_off
            return carry

        lax.fori_loop(0, RPW // 16, addoff, 0)

        def gd(c, b):
            return pltpu.make_async_copy(
                x_hbm.at[idx_all.at[pl.ds(c * CH, CH)]], rows[b], gsem[b])

        def wd(c, b):
            return pltpu.make_async_copy(
                rows[b], out_hbm.at[pl.ds(base + c * CH, CH)], wsem[b])

        for b in range(NBUF):
            gd(b, b).start()

        def body(g, carry):
            for b in range(NBUF):
                c = g * NBUF + b
                gd(c, b).wait()
                wd(c, b).start()
                wd(c, b).wait()

                @pl.when(g < NG - 1)
                def _():
                    gd(c + NBUF, b).start()

            return carry

        lax.fori_loop(0, NG, body, 0)

    return k(x_flat, ids)


def kernel(inputs, ids_shuffle):
    x_flat = inputs.reshape(ROWS, D)
    ids = ids_shuffle.astype(jnp.int32)
    out = _sc_shuffle(x_flat, ids)
    return out.reshape(BATCH, LENGTH, D)
